# baseline (speedup 1.0000x reference)
import jax
import jax.numpy as jnp
from jax import lax
from jax.experimental import pallas as pl
from jax.experimental.pallas import tpu as pltpu
from jax.experimental.pallas import tpu_sc as plsc

N_NODES = 50000
EMB = 128
S = 10
BATCH = 10000

NC, NS = 2, 16
NW = NC * NS

C1 = 32
K0, K1 = 62, 38
KMAX = max(K0, K1)
PAIR1 = (K0 + K1) * C1
N_PAD = NS * PAIR1

C2 = 64
CHUNKS2 = 5
PER_W2 = C2 * CHUNKS2
B_PAD = NW * PER_W2

G = 128

_mesh = plsc.VectorSubcoreMesh(core_axis_name="c", subcore_axis_name="s")


def _sum_rows(rows_v, out_v, n_rows, unroll=4):
    def node(ci, _):
        r0 = ci * S
        for v in range(EMB // 16):
            sl = pl.ds(v * 16, 16)
            acc = rows_v[r0, sl]
            for sj in range(1, S):
                acc = acc + rows_v[r0 + sj, sl]
            out_v[ci, sl] = acc
        return 0

    lax.fori_loop(0, n_rows, node, 0, unroll=unroll)


def _fire_gathers(table_ref, idx_all, rows_v, goff, n_idx, sem):
    for j in range(0, n_idx, G):
        k = min(G, n_idx - j)
        pltpu.make_async_copy(
            table_ref.at[idx_all.at[pl.ds(goff + j, k)]],
            rows_v.at[pl.ds(j, k)], sem).start()


def _sc1_core(neigh_flat_ref, emb_ref, sum_ref,
              idx_all, rows, outs, gsems, osems,
              s_idx, k_chunks, chunk_off):
    base_w = s_idx * PAIR1 + chunk_off * C1
    pltpu.sync_copy(
        neigh_flat_ref.at[pl.ds(base_w * S, k_chunks * C1 * S)],
        idx_all.at[pl.ds(0, k_chunks * C1 * S)])
    _fire_gathers(emb_ref, idx_all, rows[0], 0, C1 * S, gsems[0])

    def pair(g2, _):
        for b in range(2):
            g = g2 * 2 + b
            pltpu.make_async_copy(
                emb_ref.at[pl.ds(0, C1 * S)], rows[b], gsems[b]).wait()

            @pl.when(g + 1 < k_chunks)
            def _():
                _fire_gathers(emb_ref, idx_all, rows[1 - b],
                              (g + 1) * (C1 * S), C1 * S, gsems[1 - b])

            @pl.when(g >= 2)
            def _():
                pltpu.make_async_copy(
                    outs[b], sum_ref.at[pl.ds(0, C1)], osems[b]).wait()

            _sum_rows(rows[b], outs[b], C1)
            pltpu.make_async_copy(
                outs[b], sum_ref.at[pl.ds(base_w + g * C1, C1)],
                osems[b]).start()
        return 0

    lax.fori_loop(0, k_chunks // 2, pair, 0)
    pltpu.make_async_copy(outs[0], sum_ref.at[pl.ds(0, C1)], osems[0]).wait()
    pltpu.make_async_copy(outs[1], sum_ref.at[pl.ds(0, C1)], osems[1]).wait()


def _sc1_body(neigh_flat_ref, emb_ref, sum_ref,
              idx_all, rows0, rows1, out0, out1,
              gsem0, gsem1, osem0, osem1):
    s_idx = lax.axis_index("s")
    c_idx = lax.axis_index("c")
    rows = (rows0, rows1)
    outs = (out0, out1)
    gsems = (gsem0, gsem1)
    osems = (osem0, osem1)

    @pl.when(c_idx == 0)
    def _():
        _sc1_core(neigh_flat_ref, emb_ref, sum_ref, idx_all, rows, outs,
                  gsems, osems, s_idx, K0, 0)

    @pl.when(c_idx == 1)
    def _():
        _sc1_core(neigh_flat_ref, emb_ref, sum_ref, idx_all, rows, outs,
                  gsems, osems, s_idx, K1, K0)


_sc1 = pl.kernel(
    _sc1_body,
    out_type=jax.ShapeDtypeStruct((N_PAD, EMB), jnp.float32),
    mesh=_mesh,
    scratch_types=[
        pltpu.VMEM((KMAX * C1 * S,), jnp.int32),
        pltpu.VMEM((C1 * S, EMB), jnp.float32),
        pltpu.VMEM((C1 * S, EMB), jnp.float32),
        pltpu.VMEM((C1, EMB), jnp.float32),
        pltpu.VMEM((C1, EMB), jnp.float32),
        pltpu.SemaphoreType.DMA,
        pltpu.SemaphoreType.DMA,
        pltpu.SemaphoreType.DMA,
        pltpu.SemaphoreType.DMA,
    ],
)


def _sc2_body(nb_ref, neigh128_ref, h1_ref, self_ref, sum_ref,
              nbidx_v, bn_v, rows_v, self_v, out_v, sem, sem_self, gsem):
    w = lax.axis_index("s") * NC + lax.axis_index("c")
    base_w = w * PER_W2
    pltpu.sync_copy(nb_ref.at[pl.ds(base_w, PER_W2)], nbidx_v)

    def chunk(g, _):
        nb_slice = nbidx_v.at[pl.ds(g * C2, C2)]
        pltpu.async_copy(neigh128_ref.at[nb_slice], bn_v, sem).wait()
        cp_self = pltpu.async_copy(h1_ref.at[nb_slice], self_v, sem_self)

        def fire(r, _):
            pltpu.make_async_copy(
                h1_ref.at[bn_v.at[r, pl.ds(0, S)]],
                rows_v.at[pl.ds(r * S, S)], gsem).start()
            return 0

        lax.fori_loop(0, C2, fire, 0)
        pltpu.make_async_copy(h1_ref.at[pl.ds(0, C2 * S)], rows_v, gsem).wait()
        _sum_rows(rows_v, out_v, C2)
        cp_self.wait()
        off = base_w + g * C2
        pltpu.sync_copy(self_v, self_ref.at[pl.ds(off, C2)])
        pltpu.sync_copy(out_v, sum_ref.at[pl.ds(off, C2)])
        return 0

    lax.fori_loop(0, CHUNKS2, chunk, 0)


_sc2 = pl.kernel(
    _sc2_body,
    out_type=(
        jax.ShapeDtypeStruct((B_PAD, EMB), jnp.float32),
        jax.ShapeDtypeStruct((B_PAD, EMB), jnp.float32),
    ),
    mesh=_mesh,
    scratch_types=[
        pltpu.VMEM((PER_W2,), jnp.int32),
        pltpu.VMEM((C2, EMB), jnp.int32),
        pltpu.VMEM((C2 * S, EMB), jnp.float32),
        pltpu.VMEM((C2, EMB), jnp.float32),
        pltpu.VMEM((C2, EMB), jnp.float32),
        pltpu.SemaphoreType.DMA,
        pltpu.SemaphoreType.DMA,
        pltpu.SemaphoreType.DMA,
    ],
)


def _mm_body(x_ref, a_ref, wx_ref, wa_ref, o_ref):
    o_ref[...] = jnp.maximum(
        jnp.dot(x_ref[...], wx_ref[...], preferred_element_type=jnp.float32)
        + jnp.dot(a_ref[...], wa_ref[...], preferred_element_type=jnp.float32),
        0.0,
    )


def _dense(x, a, wx, wa, rows, blk):
    return pl.pallas_call(
        _mm_body,
        grid=(rows // blk,),
        in_specs=[
            pl.BlockSpec((blk, EMB), lambda i: (i, 0)),
            pl.BlockSpec((blk, EMB), lambda i: (i, 0)),
            pl.BlockSpec((EMB, EMB), lambda i: (0, 0)),
            pl.BlockSpec((EMB, EMB), lambda i: (0, 0)),
        ],
        out_specs=pl.BlockSpec((blk, EMB), lambda i: (i, 0)),
        out_shape=jax.ShapeDtypeStruct((rows, EMB), jnp.float32),
    )(x, a, wx, wa)


def kernel(node_batch, neigh_idx, emb, W1, W2):
    neigh_pad = jnp.pad(neigh_idx, ((0, N_PAD - N_NODES), (0, 0)))
    neigh_flat = neigh_pad.reshape(N_PAD * S)
    neigh128 = jnp.pad(neigh_idx, ((0, 0), (0, EMB - S)))
    nb_pad = jnp.pad(node_batch, (0, B_PAD - BATCH))

    sum1 = _sc1(neigh_flat, emb)
    W1xt = W1[:, :EMB].T
    W1at = W1[:, EMB:].T * (1.0 / S)
    h1 = _dense(emb, sum1, W1xt, W1at, N_NODES, 400)

    self2, sum2 = _sc2(nb_pad, neigh128, h1)
    W2xt = W2[:, :EMB].T
    W2at = W2[:, EMB:].T * (1.0 / S)
    return _dense(self2, sum2, W2xt, W2at, BATCH, 400)


# speedup vs baseline: 1.1286x; 1.1286x over previous
import jax
import jax.numpy as jnp
from jax import lax
from jax.experimental import pallas as pl
from jax.experimental.pallas import tpu as pltpu
from jax.experimental.pallas import tpu_sc as plsc

N_NODES = 50000
EMB = 128
S = 10
BATCH = 10000

NC, NS = 2, 16
NW = NC * NS

C1 = 32
K0, K1 = 76, 24
KMAX = max(K0, K1)
PAIR1 = (K0 + K1) * C1
N_PAD = NS * PAIR1

C2 = 64
CHUNKS2 = 5
PER_W2 = C2 * CHUNKS2
B_PAD = NW * PER_W2

G = 128

_mesh = plsc.VectorSubcoreMesh(core_axis_name="c", subcore_axis_name="s")


def _sum_rows(rows_v, out_v, n_rows, unroll=4):
    def node(ci, _):
        r0 = ci * S
        for v in range(EMB // 16):
            sl = pl.ds(v * 16, 16)
            acc = rows_v[r0, sl]
            for sj in range(1, S):
                acc = acc + rows_v[r0 + sj, sl]
            out_v[ci, sl] = acc
        return 0

    lax.fori_loop(0, n_rows, node, 0, unroll=unroll)


def _fire_gathers(table_ref, idx_all, rows_v, goff, n_idx, sem):
    for j in range(0, n_idx, G):
        k = min(G, n_idx - j)
        pltpu.make_async_copy(
            table_ref.at[idx_all.at[pl.ds(goff + j, k)]],
            rows_v.at[pl.ds(j, k)], sem).start()


def _sc1_core(neigh_flat_ref, emb_ref, sum_ref,
              idx_all, rows, outs, gsems, osems,
              s_idx, k_chunks, chunk_off):
    base_w = s_idx * PAIR1 + chunk_off * C1
    pltpu.sync_copy(
        neigh_flat_ref.at[pl.ds(base_w * S, k_chunks * C1 * S)],
        idx_all.at[pl.ds(0, k_chunks * C1 * S)])
    _fire_gathers(emb_ref, idx_all, rows[0], 0, C1 * S, gsems[0])

    def pair(g2, _):
        for b in range(2):
            g = g2 * 2 + b
            pltpu.make_async_copy(
                emb_ref.at[pl.ds(0, C1 * S)], rows[b], gsems[b]).wait()

            @pl.when(g + 1 < k_chunks)
            def _():
                _fire_gathers(emb_ref, idx_all, rows[1 - b],
                              (g + 1) * (C1 * S), C1 * S, gsems[1 - b])

            @pl.when(g >= 2)
            def _():
                pltpu.make_async_copy(
                    outs[b], sum_ref.at[pl.ds(0, C1)], osems[b]).wait()

            _sum_rows(rows[b], outs[b], C1)
            pltpu.make_async_copy(
                outs[b], sum_ref.at[pl.ds(base_w + g * C1, C1)],
                osems[b]).start()
        return 0

    lax.fori_loop(0, k_chunks // 2, pair, 0)
    pltpu.make_async_copy(outs[0], sum_ref.at[pl.ds(0, C1)], osems[0]).wait()
    pltpu.make_async_copy(outs[1], sum_ref.at[pl.ds(0, C1)], osems[1]).wait()


def _sc1_body(neigh_flat_ref, emb_ref, sum_ref,
              idx_all, rows0, rows1, out0, out1,
              gsem0, gsem1, osem0, osem1):
    s_idx = lax.axis_index("s")
    c_idx = lax.axis_index("c")
    rows = (rows0, rows1)
    outs = (out0, out1)
    gsems = (gsem0, gsem1)
    osems = (osem0, osem1)

    @pl.when(c_idx == 0)
    def _():
        _sc1_core(neigh_flat_ref, emb_ref, sum_ref, idx_all, rows, outs,
                  gsems, osems, s_idx, K0, 0)

    @pl.when(c_idx == 1)
    def _():
        _sc1_core(neigh_flat_ref, emb_ref, sum_ref, idx_all, rows, outs,
                  gsems, osems, s_idx, K1, K0)


_sc1 = pl.kernel(
    _sc1_body,
    out_type=jax.ShapeDtypeStruct((N_PAD, EMB), jnp.float32),
    mesh=_mesh,
    scratch_types=[
        pltpu.VMEM((KMAX * C1 * S,), jnp.int32),
        pltpu.VMEM((C1 * S, EMB), jnp.float32),
        pltpu.VMEM((C1 * S, EMB), jnp.float32),
        pltpu.VMEM((C1, EMB), jnp.float32),
        pltpu.VMEM((C1, EMB), jnp.float32),
        pltpu.SemaphoreType.DMA,
        pltpu.SemaphoreType.DMA,
        pltpu.SemaphoreType.DMA,
        pltpu.SemaphoreType.DMA,
    ],
)


def _sc2_body(nb_ref, neigh128_ref, h1_ref, self_ref, sum_ref,
              nbidx_v, bn_v, rows_v, self_v, out_v, sem, sem_self, gsem):
    w = lax.axis_index("s") * NC + lax.axis_index("c")
    base_w = w * PER_W2
    pltpu.sync_copy(nb_ref.at[pl.ds(base_w, PER_W2)], nbidx_v)

    def chunk(g, _):
        nb_slice = nbidx_v.at[pl.ds(g * C2, C2)]
        pltpu.async_copy(neigh128_ref.at[nb_slice], bn_v, sem).wait()
        cp_self = pltpu.async_copy(h1_ref.at[nb_slice], self_v, sem_self)

        def fire(r, _):
            pltpu.make_async_copy(
                h1_ref.at[bn_v.at[r, pl.ds(0, S)]],
                rows_v.at[pl.ds(r * S, S)], gsem).start()
            return 0

        lax.fori_loop(0, C2, fire, 0)
        pltpu.make_async_copy(h1_ref.at[pl.ds(0, C2 * S)], rows_v, gsem).wait()
        _sum_rows(rows_v, out_v, C2)
        cp_self.wait()
        off = base_w + g * C2
        pltpu.sync_copy(self_v, self_ref.at[pl.ds(off, C2)])
        pltpu.sync_copy(out_v, sum_ref.at[pl.ds(off, C2)])
        return 0

    lax.fori_loop(0, CHUNKS2, chunk, 0)


_sc2 = pl.kernel(
    _sc2_body,
    out_type=(
        jax.ShapeDtypeStruct((B_PAD, EMB), jnp.float32),
        jax.ShapeDtypeStruct((B_PAD, EMB), jnp.float32),
    ),
    mesh=_mesh,
    scratch_types=[
        pltpu.VMEM((PER_W2,), jnp.int32),
        pltpu.VMEM((C2, EMB), jnp.int32),
        pltpu.VMEM((C2 * S, EMB), jnp.float32),
        pltpu.VMEM((C2, EMB), jnp.float32),
        pltpu.VMEM((C2, EMB), jnp.float32),
        pltpu.SemaphoreType.DMA,
        pltpu.SemaphoreType.DMA,
        pltpu.SemaphoreType.DMA,
    ],
)


def _mm_body(x_ref, a_ref, wx_ref, wa_ref, o_ref):
    o_ref[...] = jnp.maximum(
        jnp.dot(x_ref[...], wx_ref[...], preferred_element_type=jnp.float32)
        + jnp.dot(a_ref[...], wa_ref[...], preferred_element_type=jnp.float32),
        0.0,
    )


def _dense(x, a, wx, wa, rows, blk):
    return pl.pallas_call(
        _mm_body,
        grid=(rows // blk,),
        in_specs=[
            pl.BlockSpec((blk, EMB), lambda i: (i, 0)),
            pl.BlockSpec((blk, EMB), lambda i: (i, 0)),
            pl.BlockSpec((EMB, EMB), lambda i: (0, 0)),
            pl.BlockSpec((EMB, EMB), lambda i: (0, 0)),
        ],
        out_specs=pl.BlockSpec((blk, EMB), lambda i: (i, 0)),
        out_shape=jax.ShapeDtypeStruct((rows, EMB), jnp.float32),
    )(x, a, wx, wa)


def kernel(node_batch, neigh_idx, emb, W1, W2):
    neigh_pad = jnp.pad(neigh_idx, ((0, N_PAD - N_NODES), (0, 0)))
    neigh_flat = neigh_pad.reshape(N_PAD * S)
    neigh128 = jnp.pad(neigh_idx, ((0, 0), (0, EMB - S)))
    nb_pad = jnp.pad(node_batch, (0, B_PAD - BATCH))

    sum1 = _sc1(neigh_flat, emb)
    W1xt = W1[:, :EMB].T
    W1at = W1[:, EMB:].T * (1.0 / S)
    h1 = _dense(emb, sum1, W1xt, W1at, N_NODES, 400)

    self2, sum2 = _sc2(nb_pad, neigh128, h1)
    W2xt = W2[:, :EMB].T
    W2at = W2[:, EMB:].T * (1.0 / S)
    return _dense(self2, sum2, W2xt, W2at, BATCH, 400)
